# Initial kernel scaffold; baseline (speedup 1.0000x reference)
import jax
import jax.numpy as jnp
from jax.experimental import pallas as pl


def kernel(boxes, mount):
    raise NotImplementedError("write your pallas kernel here")


# speedup vs baseline: 5.4657x; 5.4657x over previous
import functools

import jax
import jax.numpy as jnp
from jax import lax
from jax.experimental import pallas as pl
from jax.experimental.pallas import tpu as pltpu
from jax.experimental.pallas import tpu_sc as plsc

W = 2048
H = 2048
NB = 20000

_LANES = 16
_GROUPS = NB // _LANES
_NC = 2
_NS = 16
_NW = _NC * _NS
_ROWS_PER_W = H // _NW
_BAND = 8
_PASSES = _ROWS_PER_W // _BAND


def _sc_body(cxy_hbm, w_hbm, h_hbm, mnt_hbm,
             heat_out, size_out,
             cxy_v, w_v, h_v, mnt_v, bh, b0, b1):
    cid = lax.axis_index("c")
    sid = lax.axis_index("s")
    wid = sid * _NC + cid

    pltpu.sync_copy(cxy_hbm, cxy_v)
    pltpu.sync_copy(w_hbm, w_v)
    pltpu.sync_copy(h_hbm, h_v)
    pltpu.sync_copy(mnt_hbm, mnt_v)

    msplat = [mnt_v[j] for j in range(9)]

    zeros = jnp.zeros((_LANES,), jnp.float32)
    base_row = wid * _ROWS_PER_W

    def band_pass(p, carry):
        r0 = base_row + p * _BAND

        def zero_body(i, c):
            r = i // (W // _LANES)
            off = (i % (W // _LANES)) * _LANES
            bh[r, pl.ds(off, _LANES)] = zeros
            b0[r, pl.ds(off, _LANES)] = zeros
            b1[r, pl.ds(off, _LANES)] = zeros
            return c
        lax.fori_loop(0, _BAND * (W // _LANES), zero_body, 0)

        def group_body(g, c):
            cxy = cxy_v[pl.ds(g * _LANES, _LANES)]
            cy = lax.shift_right_logical(cxy, 11)
            row = cy - r0
            act = (row >= -1) & (row <= _BAND)

            @pl.when(jnp.any(act))
            def _():
                cx = cxy & (W - 1)
                wv = w_v[pl.ds(g * _LANES, _LANES)]
                hv = h_v[pl.ds(g * _LANES, _LANES)]

                s_ok = (row >= 0) & (row < _BAND)
                srow = jnp.where(s_ok, row, 0)
                plsc.store_scatter(b0, [srow, cx], wv, mask=s_ok)
                plsc.store_scatter(b1, [srow, cx], hv, mask=s_ok)

                valid = ((cx >= 1) & (cx <= W - 2) &
                         (cy >= 1) & (cy <= H - 2))
                for j, (dy, dx) in enumerate(
                        [(dy, dx) for dy in (-1, 0, 1) for dx in (-1, 0, 1)]):
                    hrow = row + dy
                    ok = valid & (hrow >= 0) & (hrow < _BAND)
                    hr = jnp.where(ok, hrow, 0)
                    hc = jnp.where(ok, cx + dx, 0)
                    old = plsc.load_gather(bh, [hr, hc], mask=ok)
                    new = jnp.maximum(old, msplat[j])
                    plsc.store_scatter(bh, [hr, hc], new, mask=ok)
            return c

        lax.fori_loop(0, _GROUPS, group_body, 0)

        pltpu.sync_copy(bh, heat_out.at[pl.ds(r0, _BAND)])
        pltpu.sync_copy(b0, size_out.at[0, pl.ds(r0, _BAND)])
        pltpu.sync_copy(b1, size_out.at[1, pl.ds(r0, _BAND)])
        return carry

    lax.fori_loop(0, _PASSES, band_pass, 0)


@jax.jit
def _sc_call(cxy, wv, hv, mnt):
    f = pl.kernel(
        _sc_body,
        out_type=[
            jax.ShapeDtypeStruct((H, W), jnp.float32),
            jax.ShapeDtypeStruct((2, H, W), jnp.float32),
        ],
        mesh=plsc.VectorSubcoreMesh(core_axis_name="c", subcore_axis_name="s"),
        compiler_params=pltpu.CompilerParams(needs_layout_passes=False),
        scratch_types=[
            pltpu.VMEM((NB,), jnp.int32),
            pltpu.VMEM((NB,), jnp.float32),
            pltpu.VMEM((NB,), jnp.float32),
            pltpu.VMEM((9, _LANES), jnp.float32),
            pltpu.VMEM((_BAND, W), jnp.float32),
            pltpu.VMEM((_BAND, W), jnp.float32),
            pltpu.VMEM((_BAND, W), jnp.float32),
        ],
    )
    return f(cxy, wv, hv, mnt)


def kernel(boxes, mount):
    cxi = (boxes[:, 0] * W).astype(jnp.int32)
    cyi = (boxes[:, 1] * H).astype(jnp.int32)
    cxy = cyi * W + cxi
    wv = boxes[:, 2]
    hv = boxes[:, 3]
    mnt = jnp.broadcast_to(mount.reshape(9, 1), (9, _LANES))
    heat, size = _sc_call(cxy, wv, hv, mnt)
    return heat.reshape(1, 1, H, W), size.reshape(1, 2, H, W)


# speedup vs baseline: 25.5720x; 4.6787x over previous
import functools

import jax
import jax.numpy as jnp
from jax import lax
from jax.experimental import pallas as pl
from jax.experimental.pallas import tpu as pltpu
from jax.experimental.pallas import tpu_sc as plsc

W = 2048
H = 2048
NB = 20000

_LANES = 16
_GROUPS = NB // _LANES
_NC = 2
_NS = 16
_NW = _NC * _NS
_ROWS_PER_W = H // _NW
_BAND = 8
_PASSES = _ROWS_PER_W // _BAND
_CAP = 1280

_OFFS = [(dy, dx) for dy in (-1, 0, 1) for dx in (-1, 0, 1)]


def _sc_body(cxy_hbm, w_hbm, h_hbm, mnt_hbm,
             heat_out, size_out,
             cxy_v, w_v, h_v, mnt_v,
             mcxy, mw, mh, bcxy, bw, bhv,
             hb, s0b, s1b):
    cid = lax.axis_index("c")
    sid = lax.axis_index("s")
    wid = sid * _NC + cid

    pltpu.sync_copy(cxy_hbm, cxy_v)
    pltpu.sync_copy(w_hbm, w_v)
    pltpu.sync_copy(h_hbm, h_v)
    pltpu.sync_copy(mnt_hbm, mnt_v)

    msplat = [mnt_v[j] for j in range(9)]

    zeros = jnp.zeros((_LANES,), jnp.float32)
    lane = lax.iota(jnp.int32, _LANES)
    region0 = wid * _ROWS_PER_W

    def zero_body(i, c):
        r = i // (W // _LANES)
        off = (i % (W // _LANES)) * _LANES
        hb[r, pl.ds(off, _LANES)] = zeros
        s0b[r, pl.ds(off, _LANES)] = zeros
        s1b[r, pl.ds(off, _LANES)] = zeros
        return c
    lax.fori_loop(0, _BAND * (W // _LANES), zero_body, 0)

    def l1_body(g, cnt):
        cxy = cxy_v[pl.ds(g * _LANES, _LANES)]
        r = lax.shift_right_logical(cxy, 11) - region0
        m = (r >= -1) & (r <= _ROWS_PER_W)
        at = jnp.minimum(cnt, _CAP - _LANES)
        plsc.store_compressed(mcxy.at[pl.ds(at, _LANES)], cxy, mask=m)
        plsc.store_compressed(mw.at[pl.ds(at, _LANES)],
                              w_v[pl.ds(g * _LANES, _LANES)], mask=m)
        plsc.store_compressed(mh.at[pl.ds(at, _LANES)],
                              h_v[pl.ds(g * _LANES, _LANES)], mask=m)
        return cnt + jnp.sum(m.astype(jnp.int32))
    mcnt = lax.fori_loop(0, _GROUPS, l1_body, 0)
    mgroups = (mcnt + _LANES - 1) // _LANES

    def band_pass(p, carry):
        r0 = region0 + p * _BAND

        def l2_body(g, cnt):
            tail = lane < (mcnt - g * _LANES)
            cxy = mcxy[pl.ds(g * _LANES, _LANES)]
            cy = lax.shift_right_logical(cxy, 11)
            m = tail & (cy >= r0 - 1) & (cy <= r0 + _BAND)
            at = jnp.minimum(cnt, _CAP - _LANES)
            plsc.store_compressed(bcxy.at[pl.ds(at, _LANES)], cxy, mask=m)
            plsc.store_compressed(bw.at[pl.ds(at, _LANES)],
                                  mw[pl.ds(g * _LANES, _LANES)], mask=m)
            plsc.store_compressed(bhv.at[pl.ds(at, _LANES)],
                                  mh[pl.ds(g * _LANES, _LANES)], mask=m)
            return cnt + jnp.sum(m.astype(jnp.int32))
        bcnt = lax.fori_loop(0, mgroups, l2_body, 0)

        def stamp(g, c):
            tail = lane < (bcnt - g * _LANES)
            cxy = bcxy[pl.ds(g * _LANES, _LANES)]
            cy = lax.shift_right_logical(cxy, 11)
            cx = cxy & (W - 1)
            row = cy - r0

            s_ok = tail & (row >= 0) & (row < _BAND)
            srow = jnp.where(s_ok, row, 0)
            plsc.store_scatter(s0b, [srow, cx],
                               bw[pl.ds(g * _LANES, _LANES)], mask=s_ok)
            plsc.store_scatter(s1b, [srow, cx],
                               bhv[pl.ds(g * _LANES, _LANES)], mask=s_ok)

            valid = (tail & (cx >= 1) & (cx <= W - 2) &
                     (cy >= 1) & (cy <= H - 2))
            for j, (dy, dx) in enumerate(_OFFS):
                hrow = row + dy
                ok = valid & (hrow >= 0) & (hrow < _BAND)
                hr = jnp.where(ok, hrow, 0)
                hc = jnp.where(ok, cx + dx, 0)
                old = plsc.load_gather(hb, [hr, hc], mask=ok)
                plsc.store_scatter(hb, [hr, hc],
                                   jnp.maximum(old, msplat[j]), mask=ok)
            return c
        bgroups = (bcnt + _LANES - 1) // _LANES
        lax.fori_loop(0, bgroups, stamp, 0)

        pltpu.sync_copy(hb, heat_out.at[pl.ds(r0, _BAND)])
        pltpu.sync_copy(s0b, size_out.at[0, pl.ds(r0, _BAND)])
        pltpu.sync_copy(s1b, size_out.at[1, pl.ds(r0, _BAND)])

        def clear(g, c):
            tail = lane < (bcnt - g * _LANES)
            cxy = bcxy[pl.ds(g * _LANES, _LANES)]
            cy = lax.shift_right_logical(cxy, 11)
            cx = cxy & (W - 1)
            row = cy - r0
            s_ok = tail & (row >= 0) & (row < _BAND)
            srow = jnp.where(s_ok, row, 0)
            plsc.store_scatter(s0b, [srow, cx], zeros, mask=s_ok)
            plsc.store_scatter(s1b, [srow, cx], zeros, mask=s_ok)
            for dy, dx in ((-1, 0), (0, 0), (1, 0)):
                hrow = row + dy
                ok = tail & (hrow >= 0) & (hrow < _BAND)
                hr = jnp.where(ok, hrow, 0)
                for dx2 in (-1, 0, 1):
                    hc = jnp.clip(cx + dx2, 0, W - 1)
                    plsc.store_scatter(hb, [hr, hc], zeros, mask=ok)
            return c
        lax.fori_loop(0, bgroups, clear, 0)
        return carry

    lax.fori_loop(0, _PASSES, band_pass, 0)


@jax.jit
def _sc_call(cxy, wv, hv, mnt):
    f = pl.kernel(
        _sc_body,
        out_type=[
            jax.ShapeDtypeStruct((H, W), jnp.float32),
            jax.ShapeDtypeStruct((2, H, W), jnp.float32),
        ],
        mesh=plsc.VectorSubcoreMesh(core_axis_name="c", subcore_axis_name="s"),
        compiler_params=pltpu.CompilerParams(needs_layout_passes=False),
        scratch_types=[
            pltpu.VMEM((NB,), jnp.int32),
            pltpu.VMEM((NB,), jnp.float32),
            pltpu.VMEM((NB,), jnp.float32),
            pltpu.VMEM((9, _LANES), jnp.float32),
            pltpu.VMEM((_CAP,), jnp.int32),
            pltpu.VMEM((_CAP,), jnp.float32),
            pltpu.VMEM((_CAP,), jnp.float32),
            pltpu.VMEM((_CAP,), jnp.int32),
            pltpu.VMEM((_CAP,), jnp.float32),
            pltpu.VMEM((_CAP,), jnp.float32),
            pltpu.VMEM((_BAND, W), jnp.float32),
            pltpu.VMEM((_BAND, W), jnp.float32),
            pltpu.VMEM((_BAND, W), jnp.float32),
        ],
    )
    return f(cxy, wv, hv, mnt)


def kernel(boxes, mount):
    cxi = (boxes[:, 0] * W).astype(jnp.int32)
    cyi = (boxes[:, 1] * H).astype(jnp.int32)
    cxy = cyi * W + cxi
    wv = boxes[:, 2]
    hv = boxes[:, 3]
    mnt = jnp.broadcast_to(mount.reshape(9, 1), (9, _LANES))
    heat, size = _sc_call(cxy, wv, hv, mnt)
    return heat.reshape(1, 1, H, W), size.reshape(1, 2, H, W)


# speedup vs baseline: 28.5607x; 1.1169x over previous
import functools

import jax
import jax.numpy as jnp
from jax import lax
from jax.experimental import pallas as pl
from jax.experimental.pallas import tpu as pltpu
from jax.experimental.pallas import tpu_sc as plsc

W = 2048
H = 2048
NB = 20000

_LANES = 16
_GROUPS = NB // _LANES
_NC = 2
_NS = 16
_NW = _NC * _NS
_ROWS_PER_W = H // _NW
_BAND = 4
_PASSES = _ROWS_PER_W // _BAND
_CAP = 1280

_OFFS = [(dy, dx) for dy in (-1, 0, 1) for dx in (-1, 0, 1)]


def _sc_body(cxy_hbm, w_hbm, h_hbm, mnt_hbm,
             heat_out, size_out,
             cxy_v, w_v, h_v, mnt_v,
             mcxy, mw, mh, bw, bhv,
             bcxy_a, bcxy_b, hb_a, hb_b, s0_a, s0_b, s1_a, s1_b,
             cnt_s, sem):
    cid = lax.axis_index("c")
    sid = lax.axis_index("s")
    wid = sid * _NC + cid

    pltpu.sync_copy(cxy_hbm, cxy_v)
    pltpu.sync_copy(w_hbm, w_v)
    pltpu.sync_copy(h_hbm, h_v)
    pltpu.sync_copy(mnt_hbm, mnt_v)

    msplat = [mnt_v[j] for j in range(9)]

    zeros = jnp.zeros((_LANES,), jnp.float32)
    lane = lax.iota(jnp.int32, _LANES)
    region0 = wid * _ROWS_PER_W
    cnt_s[0] = 0
    cnt_s[1] = 0

    def zero_body(i, c):
        r = i // (W // _LANES)
        off = (i % (W // _LANES)) * _LANES
        for ref in (hb_a, hb_b, s0_a, s0_b, s1_a, s1_b):
            ref[r, pl.ds(off, _LANES)] = zeros
        return c
    lax.fori_loop(0, _BAND * (W // _LANES), zero_body, 0)

    def l1_body(g, cnt):
        cxy = cxy_v[pl.ds(g * _LANES, _LANES)]
        r = lax.shift_right_logical(cxy, 11) - region0
        m = (r >= -1) & (r <= _ROWS_PER_W)
        at = jnp.minimum(cnt, _CAP - _LANES)
        plsc.store_compressed(mcxy.at[pl.ds(at, _LANES)], cxy, mask=m)
        plsc.store_compressed(mw.at[pl.ds(at, _LANES)],
                              w_v[pl.ds(g * _LANES, _LANES)], mask=m)
        plsc.store_compressed(mh.at[pl.ds(at, _LANES)],
                              h_v[pl.ds(g * _LANES, _LANES)], mask=m)
        return cnt + jnp.sum(m.astype(jnp.int32))
    mcnt = lax.fori_loop(0, _GROUPS, l1_body, 0)
    mgroups = (mcnt + _LANES - 1) // _LANES

    def do_pass(p, parity, bcxy, hb, s0b, s1b):
        r0 = region0 + p * _BAND
        r_old = r0 - 2 * _BAND

        @pl.when(p >= 2)
        def _():
            pltpu.make_async_copy(hb, heat_out.at[pl.ds(r_old, _BAND)],
                                  sem).wait()
            pltpu.make_async_copy(s0b, size_out.at[0, pl.ds(r_old, _BAND)],
                                  sem).wait()
            pltpu.make_async_copy(s1b, size_out.at[1, pl.ds(r_old, _BAND)],
                                  sem).wait()

        oc = cnt_s[parity]

        def clear(g, c):
            tail = lane < (oc - g * _LANES)
            cxy = bcxy[pl.ds(g * _LANES, _LANES)]
            cy = lax.shift_right_logical(cxy, 11)
            cx = cxy & (W - 1)
            row = cy - r_old
            s_ok = tail & (row >= 0) & (row < _BAND)
            srow = jnp.where(s_ok, row, 0)
            plsc.store_scatter(s0b, [srow, cx], zeros, mask=s_ok)
            plsc.store_scatter(s1b, [srow, cx], zeros, mask=s_ok)
            for dy in (-1, 0, 1):
                hrow = row + dy
                ok = tail & (hrow >= 0) & (hrow < _BAND)
                hr = jnp.where(ok, hrow, 0)
                for dx in (-1, 0, 1):
                    hc = jnp.clip(cx + dx, 0, W - 1)
                    plsc.store_scatter(hb, [hr, hc], zeros, mask=ok)
            return c
        lax.fori_loop(0, (oc + _LANES - 1) // _LANES, clear, 0)

        def l2_body(g, cnt):
            tail = lane < (mcnt - g * _LANES)
            cxy = mcxy[pl.ds(g * _LANES, _LANES)]
            cy = lax.shift_right_logical(cxy, 11)
            m = tail & (cy >= r0 - 1) & (cy <= r0 + _BAND)
            at = jnp.minimum(cnt, _CAP - _LANES)
            plsc.store_compressed(bcxy.at[pl.ds(at, _LANES)], cxy, mask=m)
            plsc.store_compressed(bw.at[pl.ds(at, _LANES)],
                                  mw[pl.ds(g * _LANES, _LANES)], mask=m)
            plsc.store_compressed(bhv.at[pl.ds(at, _LANES)],
                                  mh[pl.ds(g * _LANES, _LANES)], mask=m)
            return cnt + jnp.sum(m.astype(jnp.int32))
        bcnt = lax.fori_loop(0, mgroups, l2_body, 0)
        cnt_s[parity] = bcnt

        def stamp(g, c):
            tail = lane < (bcnt - g * _LANES)
            cxy = bcxy[pl.ds(g * _LANES, _LANES)]
            cy = lax.shift_right_logical(cxy, 11)
            cx = cxy & (W - 1)
            row = cy - r0

            s_ok = tail & (row >= 0) & (row < _BAND)
            srow = jnp.where(s_ok, row, 0)
            plsc.store_scatter(s0b, [srow, cx],
                               bw[pl.ds(g * _LANES, _LANES)], mask=s_ok)
            plsc.store_scatter(s1b, [srow, cx],
                               bhv[pl.ds(g * _LANES, _LANES)], mask=s_ok)

            valid = (tail & (cx >= 1) & (cx <= W - 2) &
                     (cy >= 1) & (cy <= H - 2))
            for j, (dy, dx) in enumerate(_OFFS):
                hrow = row + dy
                ok = valid & (hrow >= 0) & (hrow < _BAND)
                hr = jnp.where(ok, hrow, 0)
                hc = jnp.where(ok, cx + dx, 0)
                old = plsc.load_gather(hb, [hr, hc], mask=ok)
                plsc.store_scatter(hb, [hr, hc],
                                   jnp.maximum(old, msplat[j]), mask=ok)
            return c
        lax.fori_loop(0, (bcnt + _LANES - 1) // _LANES, stamp, 0)

        pltpu.async_copy(hb, heat_out.at[pl.ds(r0, _BAND)], sem)
        pltpu.async_copy(s0b, size_out.at[0, pl.ds(r0, _BAND)], sem)
        pltpu.async_copy(s1b, size_out.at[1, pl.ds(r0, _BAND)], sem)

    def band_pass(p, carry):
        @pl.when(p % 2 == 0)
        def _():
            do_pass(p, 0, bcxy_a, hb_a, s0_a, s1_a)

        @pl.when(p % 2 == 1)
        def _():
            do_pass(p, 1, bcxy_b, hb_b, s0_b, s1_b)
        return carry

    lax.fori_loop(0, _PASSES, band_pass, 0)

    for p, (hb, s0b, s1b) in ((_PASSES - 2, (hb_a, s0_a, s1_a)),
                              (_PASSES - 1, (hb_b, s0_b, s1_b))):
        r0 = region0 + p * _BAND
        pltpu.make_async_copy(hb, heat_out.at[pl.ds(r0, _BAND)], sem).wait()
        pltpu.make_async_copy(s0b, size_out.at[0, pl.ds(r0, _BAND)],
                              sem).wait()
        pltpu.make_async_copy(s1b, size_out.at[1, pl.ds(r0, _BAND)],
                              sem).wait()


@jax.jit
def _sc_call(cxy, wv, hv, mnt):
    f = pl.kernel(
        _sc_body,
        out_type=[
            jax.ShapeDtypeStruct((H, W), jnp.float32),
            jax.ShapeDtypeStruct((2, H, W), jnp.float32),
        ],
        mesh=plsc.VectorSubcoreMesh(core_axis_name="c", subcore_axis_name="s"),
        compiler_params=pltpu.CompilerParams(needs_layout_passes=False),
        scratch_types=[
            pltpu.VMEM((NB,), jnp.int32),
            pltpu.VMEM((NB,), jnp.float32),
            pltpu.VMEM((NB,), jnp.float32),
            pltpu.VMEM((9, _LANES), jnp.float32),
            pltpu.VMEM((_CAP,), jnp.int32),
            pltpu.VMEM((_CAP,), jnp.float32),
            pltpu.VMEM((_CAP,), jnp.float32),
            pltpu.VMEM((_CAP,), jnp.float32),
            pltpu.VMEM((_CAP,), jnp.float32),
            pltpu.VMEM((_CAP,), jnp.int32),
            pltpu.VMEM((_CAP,), jnp.int32),
            pltpu.VMEM((_BAND, W), jnp.float32),
            pltpu.VMEM((_BAND, W), jnp.float32),
            pltpu.VMEM((_BAND, W), jnp.float32),
            pltpu.VMEM((_BAND, W), jnp.float32),
            pltpu.VMEM((_BAND, W), jnp.float32),
            pltpu.VMEM((_BAND, W), jnp.float32),
            pltpu.SMEM((2,), jnp.int32),
            pltpu.SemaphoreType.DMA,
        ],
    )
    return f(cxy, wv, hv, mnt)


def kernel(boxes, mount):
    cxi = (boxes[:, 0] * W).astype(jnp.int32)
    cyi = (boxes[:, 1] * H).astype(jnp.int32)
    cxy = cyi * W + cxi
    wv = boxes[:, 2]
    hv = boxes[:, 3]
    mnt = jnp.broadcast_to(mount.reshape(9, 1), (9, _LANES))
    heat, size = _sc_call(cxy, wv, hv, mnt)
    return heat.reshape(1, 1, H, W), size.reshape(1, 2, H, W)
